# baseline (speedup 1.0000x reference)
import dataclasses
import functools

import jax
import jax.numpy as jnp
from jax import lax
from jax.experimental import pallas as pl
from jax.experimental.pallas import tpu as pltpu
from jax.experimental.pallas import tpu_sc as plsc

_AH = 7
_AW = 7
_PTS = _AH * _AW
_N = 1000
_C = 256
_SIDES = (128, 64, 32, 16)
_SCALES = (0.25, 0.125, 0.0625, 0.03125)
_REF_AREAS = (1024.0, 4096.0, 16384.0, 65536.0)
_LVL_OFF = (0, 32768, 40960, 43008)
_ROWS = 43520
_PP = 56
_NR = 1024
_NQ2 = _NR * _PP
_NQR = _N * _PP
_PREP_R = 448
_PREP_C = 128

_NW = 32
_RB = _PTS * _C


def _tr_body(f_ref, t_ref):
    x = f_ref[0]
    w = x.shape[2]
    for yy in range(8):
        t_ref[pl.ds(yy * w, w), :] = x[:, yy, :].T


def _tr_call(lvl, feat, table):
    side = _SIDES[lvl]
    h8 = side // 8
    base_blk = _LVL_OFF[lvl] // (8 * side)
    args = [feat]
    io_alias = {}
    in_specs = [pl.BlockSpec((1, _C, 8, side),
                             lambda b, yb: (b, 0, yb, 0))]
    body = _tr_body
    if table is not None:
        args = [table, feat]
        io_alias = {0: 0}
        in_specs = [pl.BlockSpec(memory_space=pl.ANY)] + in_specs
        body = lambda _tab, f_ref, t_ref: _tr_body(f_ref, t_ref)
    grid = (feat.shape[0], h8)
    return pl.pallas_call(
        body,
        grid=grid,
        in_specs=in_specs,
        out_specs=pl.BlockSpec(
            (8 * side, _C),
            lambda b, yb, _h8=h8, _bb=base_blk: (_bb + b * _h8 + yb, 0)),
        out_shape=jax.ShapeDtypeStruct((_ROWS, _C), jnp.float32),
        input_output_aliases=io_alias,
    )(*args)


def _build_table(feats):
    table = None
    for lvl, f in enumerate(feats):
        table = _tr_call(lvl, f, table)
    return table


def _prep_body(x1r, y1r, x2r, y2r, br,
               i0, i1, i2, i3, j0, j1, j2, j3, w0, w1, w2, w3):
    x1 = x1r[...]
    y1 = y1r[...]
    x2 = x2r[...]
    y2 = y2r[...]
    b = br[...]
    rows = lax.broadcasted_iota(jnp.int32, (_PREP_R, _PREP_C), 0)
    cols = lax.broadcasted_iota(jnp.int32, (_PREP_R, _PREP_C), 1)
    q = rows * _PREP_C + cols
    p = q % _PP
    py = p // _AW
    px = p % _AW

    area = (x2 - x1 + 1.0) * (y2 - y1 + 1.0)
    lvl = jnp.zeros_like(q)
    best = jnp.abs(jnp.log(jnp.sqrt(area / _REF_AREAS[0])) / 0.6931472)
    for l in range(1, 4):
        v = jnp.abs(jnp.log(jnp.sqrt(area / _REF_AREAS[l])) / 0.6931472)
        upd = v < best
        lvl = jnp.where(upd, l, lvl)
        best = jnp.where(upd, v, best)

    def sel_f(vals):
        return jnp.where(lvl == 0, vals[0],
                         jnp.where(lvl == 1, vals[1],
                                   jnp.where(lvl == 2, vals[2], vals[3])))

    scale = sel_f([jnp.float32(s) for s in _SCALES])
    side_f = sel_f([jnp.float32(s) for s in _SIDES])
    side_i = sel_f([jnp.int32(s) for s in _SIDES])
    off = sel_f([jnp.int32(s) for s in _LVL_OFF])
    hw = side_i * side_i

    x1s = x1 * scale
    y1s = y1 * scale
    x2s = x2 * scale
    y2s = y2 * scale
    roi_w = jnp.maximum(x2s - x1s, 1.0)
    roi_h = jnp.maximum(y2s - y1s, 1.0)
    bin_w = roi_w / _AW
    bin_h = roi_h / _AH
    sx = x1s + bin_w * (px.astype(jnp.float32) + 0.5)
    sy = y1s + bin_h * (py.astype(jnp.float32) + 0.5)
    valid = (sy > -1.0) & (sy < side_f) & (sx > -1.0) & (sx < side_f)
    yc = jnp.clip(sy, 0.0, side_f - 1.0)
    xc = jnp.clip(sx, 0.0, side_f - 1.0)
    y0f = jnp.floor(yc)
    x0f = jnp.floor(xc)
    y0 = y0f.astype(jnp.int32)
    x0 = x0f.astype(jnp.int32)
    y1i = jnp.minimum(y0 + 1, side_i - 1)
    x1i = jnp.minimum(x0 + 1, side_i - 1)
    ly = yc - y0f
    lx = xc - x0f
    hy = 1.0 - ly
    hx = 1.0 - lx
    vm = (valid & (p < _PTS) & (q < _NQR)).astype(jnp.float32)

    w0[...] = hy * hx * vm
    w1[...] = hy * lx * vm
    w2[...] = ly * hx * vm
    w3[...] = ly * lx * vm
    rb = off + b * hw
    e0 = (rb + y0 * side_i + x0) * 2
    e1 = (rb + y0 * side_i + x1i) * 2
    e2 = (rb + y1i * side_i + x0) * 2
    e3 = (rb + y1i * side_i + x1i) * 2
    i0[...] = e0
    i1[...] = e1
    i2[...] = e2
    i3[...] = e3
    j0[...] = e0 + 1
    j1[...] = e1 + 1
    j2[...] = e2 + 1
    j3[...] = e3 + 1


def _prep_call(x1r, y1r, x2r, y2r, br):
    i32 = jax.ShapeDtypeStruct((_PREP_R, _PREP_C), jnp.int32)
    f32 = jax.ShapeDtypeStruct((_PREP_R, _PREP_C), jnp.float32)
    return pl.pallas_call(
        _prep_body,
        out_shape=[i32, i32, i32, i32, i32, i32, i32, i32,
                   f32, f32, f32, f32],
    )(x1r, y1r, x2r, y2r, br)


def _sc_body(tab, i0, i1, i2, i3, j0, j1, j2, j3, w0, w1, w2, w3, out,
             iv0, iv1, iv2, iv3, jv0, jv1, jv2, jv3, wv0, wv1, wv2, wv3,
             ra0, ra1, ra2, ra3, rb0, rb1, rb2, rb3, acc,
             gs0, gs1, ss):
    wid = lax.axis_index("s") * 2 + lax.axis_index("c")
    nr = jnp.where(wid < 8, 32, 31)
    base_roi = wid * 31 + jnp.minimum(wid, 8)
    sl = pl.ds(base_roi * _PP, 32 * _PP)
    cps = [
        pltpu.async_copy(i0.at[sl], iv0, gs0),
        pltpu.async_copy(i1.at[sl], iv1, gs0),
        pltpu.async_copy(i2.at[sl], iv2, gs0),
        pltpu.async_copy(i3.at[sl], iv3, gs0),
        pltpu.async_copy(j0.at[sl], jv0, gs0),
        pltpu.async_copy(j1.at[sl], jv1, gs0),
        pltpu.async_copy(j2.at[sl], jv2, gs0),
        pltpu.async_copy(j3.at[sl], jv3, gs0),
        pltpu.async_copy(w0.at[sl], wv0, gs0),
        pltpu.async_copy(w1.at[sl], wv1, gs0),
        pltpu.async_copy(w2.at[sl], wv2, gs0),
        pltpu.async_copy(w3.at[sl], wv3, gs0),
    ]
    for cp in cps:
        cp.wait()

    ivsets = ((iv0, iv1, iv2, iv3), (jv0, jv1, jv2, jv3))
    rsets = ((ra0, ra1, ra2, ra3), (rb0, rb1, rb2, rb3))
    gsems = (gs0, gs1)
    i49 = lax.iota(jnp.int32, 16) * _PTS

    def gathers(rr, k):
        off = rr * _PP
        return [pltpu.make_async_copy(tab.at[iv.at[pl.ds(off, _PTS)]], r,
                                      gsems[k])
                for iv, r in zip(ivsets[k], rsets[k])]

    def store(rr):
        dst = out.at[pl.ds((base_roi + rr) * _RB, _RB)]
        return pltpu.make_async_copy(acc, dst, ss)

    for g in gathers(0, 0):
        g.start()
    for g in gathers(0, 1):
        g.start()

    @pl.loop(0, 64, step=2)
    def _pair(c):
        rr = c // 2
        for k in (0, 1):

            @pl.when(rr < nr)
            def _():
                rs = rsets[k]
                for g in gathers(rr, k):
                    g.wait()

                if k == 0:
                    @pl.when(rr > 0)
                    def _():
                        store(rr).wait()

                off = rr * _PP

                @pl.loop(0, _PTS)
                def _pt(p):
                    qi = jnp.full((16,), off + p, dtype=jnp.int32)
                    a0 = plsc.load_gather(wv0, [qi])
                    a1 = plsc.load_gather(wv1, [qi])
                    a2 = plsc.load_gather(wv2, [qi])
                    a3 = plsc.load_gather(wv3, [qi])
                    for j in range(8):
                        cs = pl.ds(16 * j, 16)
                        v = ((a0 * rs[0][p, cs] + a1 * rs[1][p, cs])
                             + a2 * rs[2][p, cs] + a3 * rs[3][p, cs])
                        plsc.store_scatter(
                            acc, [i49 + ((128 * k + 16 * j) * _PTS) + p], v)

                if k == 1:
                    store(rr).start()

                @pl.when(rr + 1 < nr)
                def _():
                    for g in gathers(rr + 1, k):
                        g.start()

    store(nr - 1).wait()


def _sc_call(table, i0, i1, i2, i3, j0, j1, j2, j3, w0, w1, w2, w3):
    cp = pltpu.CompilerParams()
    if "needs_layout_passes" in pltpu.CompilerParams.__dataclass_fields__:
        cp = dataclasses.replace(cp, needs_layout_passes=False)
    mesh = plsc.VectorSubcoreMesh(core_axis_name="c", subcore_axis_name="s")
    run = functools.partial(
        pl.kernel,
        out_type=jax.ShapeDtypeStruct((_N * _RB,), jnp.float32),
        mesh=mesh,
        compiler_params=cp,
        scratch_types=(
            [pltpu.VMEM((32 * _PP,), jnp.int32)] * 8
            + [pltpu.VMEM((32 * _PP,), jnp.float32)] * 4
            + [pltpu.VMEM((_PTS, 128), jnp.float32)] * 8
            + [pltpu.VMEM((_RB,), jnp.float32)]
            + [pltpu.SemaphoreType.DMA] * 3
        ),
    )(_sc_body)
    return run(table, i0, i1, i2, i3, j0, j1, j2, j3, w0, w1, w2, w3)


def _expand(col):
    e = jnp.concatenate([col, jnp.zeros((_NR - _N,), col.dtype)])
    e = jnp.repeat(e, _PP)
    return e.reshape(_PREP_R, _PREP_C)


def kernel(feat0, feat1, feat2, feat3, bboxes, batch_inds):
    feats = (feat0, feat1, feat2, feat3)
    table = jnp.concatenate(
        [jnp.transpose(f, (0, 2, 3, 1)).reshape(-1, _C) for f in feats], axis=0)

    bi = batch_inds.astype(jnp.int32)
    x1r = _expand(bboxes[:, 0])
    y1r = _expand(bboxes[:, 1])
    x2r = _expand(bboxes[:, 2])
    y2r = _expand(bboxes[:, 3])
    br = _expand(bi)

    pr = _prep_call(x1r, y1r, x2r, y2r, br)
    tab2 = table.reshape(_ROWS * 2, 128)
    out = _sc_call(tab2, *[a.reshape(_NQ2) for a in pr])
    return out.reshape(_N, _C, _AH, _AW)


# speedup vs baseline: 2.3411x; 2.3411x over previous
import dataclasses
import functools

import jax
import jax.numpy as jnp
from jax import lax
from jax.experimental import pallas as pl
from jax.experimental.pallas import tpu as pltpu
from jax.experimental.pallas import tpu_sc as plsc

_AH = 7
_AW = 7
_PTS = _AH * _AW
_N = 1000
_C = 256
_SIDES = (128, 64, 32, 16)
_SCALES = (0.25, 0.125, 0.0625, 0.03125)
_REF_AREAS = (1024.0, 4096.0, 16384.0, 65536.0)
_LVL_OFF = (0, 32768, 40960, 43008)
_ROWS = 43520
_NQ = _N * _PTS
_NP = 49152
_PREP_R = 384
_PREP_C = 128

_NW = 32
_PER_W = _NP // _NW
_G = 48
_NCH = _PER_W // _G


def _prep_body(x1r, y1r, x2r, y2r, br,
               i0, i1, i2, i3, w0, w1, w2, w3):
    x1 = x1r[...]
    y1 = y1r[...]
    x2 = x2r[...]
    y2 = y2r[...]
    b = br[...]
    rows = lax.broadcasted_iota(jnp.int32, (_PREP_R, _PREP_C), 0)
    cols = lax.broadcasted_iota(jnp.int32, (_PREP_R, _PREP_C), 1)
    q = rows * _PREP_C + cols
    p = q % _PTS
    py = p // _AW
    px = p % _AW

    area = (x2 - x1 + 1.0) * (y2 - y1 + 1.0)
    lvl = jnp.zeros_like(q)
    best = jnp.abs(jnp.log(jnp.sqrt(area / _REF_AREAS[0])) / 0.6931472)
    for l in range(1, 4):
        v = jnp.abs(jnp.log(jnp.sqrt(area / _REF_AREAS[l])) / 0.6931472)
        upd = v < best
        lvl = jnp.where(upd, l, lvl)
        best = jnp.where(upd, v, best)

    def sel_f(vals):
        return jnp.where(lvl == 0, vals[0],
                         jnp.where(lvl == 1, vals[1],
                                   jnp.where(lvl == 2, vals[2], vals[3])))

    scale = sel_f([jnp.float32(s) for s in _SCALES])
    side_f = sel_f([jnp.float32(s) for s in _SIDES])
    side_i = sel_f([jnp.int32(s) for s in _SIDES])
    off = sel_f([jnp.int32(s) for s in _LVL_OFF])
    hw = side_i * side_i

    x1s = x1 * scale
    y1s = y1 * scale
    x2s = x2 * scale
    y2s = y2 * scale
    roi_w = jnp.maximum(x2s - x1s, 1.0)
    roi_h = jnp.maximum(y2s - y1s, 1.0)
    bin_w = roi_w / _AW
    bin_h = roi_h / _AH
    sx = x1s + bin_w * (px.astype(jnp.float32) + 0.5)
    sy = y1s + bin_h * (py.astype(jnp.float32) + 0.5)
    valid = (sy > -1.0) & (sy < side_f) & (sx > -1.0) & (sx < side_f)
    yc = jnp.clip(sy, 0.0, side_f - 1.0)
    xc = jnp.clip(sx, 0.0, side_f - 1.0)
    y0f = jnp.floor(yc)
    x0f = jnp.floor(xc)
    y0 = y0f.astype(jnp.int32)
    x0 = x0f.astype(jnp.int32)
    y1i = jnp.minimum(y0 + 1, side_i - 1)
    x1i = jnp.minimum(x0 + 1, side_i - 1)
    ly = yc - y0f
    lx = xc - x0f
    hy = 1.0 - ly
    hx = 1.0 - lx
    vm = (valid & (q < _NQ)).astype(jnp.float32)

    w0[...] = hy * hx * vm
    w1[...] = hy * lx * vm
    w2[...] = ly * hx * vm
    w3[...] = ly * lx * vm
    rb = off + b * hw
    i0[...] = rb + y0 * side_i + x0
    i1[...] = rb + y0 * side_i + x1i
    i2[...] = rb + y1i * side_i + x0
    i3[...] = rb + y1i * side_i + x1i


def _prep_call(x1r, y1r, x2r, y2r, br):
    i32 = jax.ShapeDtypeStruct((_PREP_R, _PREP_C), jnp.int32)
    f32 = jax.ShapeDtypeStruct((_PREP_R, _PREP_C), jnp.float32)
    return pl.pallas_call(
        _prep_body,
        out_shape=[i32, i32, i32, i32, f32, f32, f32, f32],
    )(x1r, y1r, x2r, y2r, br)


def _sc_body(tab, i0, i1, i2, i3, w0, w1, w2, w3, out,
             iv0, iv1, iv2, iv3, wv0, wv1, wv2, wv3,
             ra0, ra1, ra2, ra3, rb0, rb1, rb2, rb3, acc,
             gs0, gs1, ss):
    wid = lax.axis_index("s") * 2 + lax.axis_index("c")
    base = wid * _PER_W
    sl = pl.ds(base, _PER_W)
    cps = [
        pltpu.async_copy(i0.at[sl], iv0, gs0),
        pltpu.async_copy(i1.at[sl], iv1, gs0),
        pltpu.async_copy(i2.at[sl], iv2, gs0),
        pltpu.async_copy(i3.at[sl], iv3, gs0),
        pltpu.async_copy(w0.at[sl], wv0, gs0),
        pltpu.async_copy(w1.at[sl], wv1, gs0),
        pltpu.async_copy(w2.at[sl], wv2, gs0),
        pltpu.async_copy(w3.at[sl], wv3, gs0),
    ]
    for cp in cps:
        cp.wait()

    ivs = (iv0, iv1, iv2, iv3)
    rsets = ((ra0, ra1, ra2, ra3), (rb0, rb1, rb2, rb3))
    gsems = (gs0, gs1)

    def gathers(cc, k):
        off = cc * _G
        return [pltpu.make_async_copy(tab.at[iv.at[pl.ds(off, _G)]], r,
                                      gsems[k])
                for iv, r in zip(ivs, rsets[k])]

    def store(cc):
        return pltpu.make_async_copy(acc, out.at[pl.ds(base + cc * _G, _G)],
                                     ss)

    for g in gathers(0, 0):
        g.start()
    for g in gathers(1, 1):
        g.start()

    @pl.loop(0, _NCH, step=2)
    def _pair(c):
        for k in (0, 1):
            cc = c + k
            rs = rsets[k]
            for g in gathers(cc, k):
                g.wait()

            @pl.when(cc > 0)
            def _():
                store(cc).wait()

            off = cc * _G

            @pl.loop(0, _G)
            def _pt(i):
                qi = jnp.full((16,), off + i, dtype=jnp.int32)
                a0 = plsc.load_gather(wv0, [qi])
                a1 = plsc.load_gather(wv1, [qi])
                a2 = plsc.load_gather(wv2, [qi])
                a3 = plsc.load_gather(wv3, [qi])
                for j in range(_C // 16):
                    cs = pl.ds(16 * j, 16)
                    acc[i, cs] = ((a0 * rs[0][i, cs] + a1 * rs[1][i, cs])
                                  + a2 * rs[2][i, cs] + a3 * rs[3][i, cs])

            store(cc).start()

            @pl.when(cc + 2 < _NCH)
            def _():
                for g in gathers(cc + 2, k):
                    g.start()

    store(_NCH - 1).wait()


def _sc_call(table, i0, i1, i2, i3, w0, w1, w2, w3):
    cp = pltpu.CompilerParams()
    if "needs_layout_passes" in pltpu.CompilerParams.__dataclass_fields__:
        cp = dataclasses.replace(cp, needs_layout_passes=False)
    mesh = plsc.VectorSubcoreMesh(core_axis_name="c", subcore_axis_name="s")
    run = functools.partial(
        pl.kernel,
        out_type=jax.ShapeDtypeStruct((_NP, _C), jnp.float32),
        mesh=mesh,
        compiler_params=cp,
        scratch_types=(
            [pltpu.VMEM((_PER_W,), jnp.int32)] * 4
            + [pltpu.VMEM((_PER_W,), jnp.float32)] * 4
            + [pltpu.VMEM((_G, _C), jnp.float32)] * 9
            + [pltpu.SemaphoreType.DMA] * 3
        ),
    )(_sc_body)
    return run(table, i0, i1, i2, i3, w0, w1, w2, w3)


def _expand(col):
    e = jnp.repeat(col, _PTS)
    e = jnp.concatenate([e, jnp.zeros((_NP - _NQ,), e.dtype)])
    return e.reshape(_PREP_R, _PREP_C)


def kernel(feat0, feat1, feat2, feat3, bboxes, batch_inds):
    feats = (feat0, feat1, feat2, feat3)
    table = jnp.concatenate(
        [jnp.transpose(f, (0, 2, 3, 1)).reshape(-1, _C) for f in feats], axis=0)

    bi = batch_inds.astype(jnp.int32)
    x1r = _expand(bboxes[:, 0])
    y1r = _expand(bboxes[:, 1])
    x2r = _expand(bboxes[:, 2])
    y2r = _expand(bboxes[:, 3])
    br = _expand(bi)

    i0, i1, i2, i3, w0, w1, w2, w3 = _prep_call(x1r, y1r, x2r, y2r, br)
    flat = lambda a: a.reshape(_NP)
    rows = _sc_call(table, flat(i0), flat(i1), flat(i2), flat(i3),
                    flat(w0), flat(w1), flat(w2), flat(w3))
    out = rows[:_NQ].reshape(_N, _PTS, _C).transpose(0, 2, 1)
    return out.reshape(_N, _C, _AH, _AW)
